# baseline (speedup 1.0000x reference)
import functools

import jax
import jax.numpy as jnp
from jax import lax
from jax.experimental import pallas as pl
from jax.experimental.pallas import tpu as pltpu
from jax.experimental.pallas import tpu_sc as plsc

SEQ = 577
NUM_HEADS = 12
HEADS_PER_CHUNK = 2
N_CHUNKS = NUM_HEADS // HEADS_PER_CHUNK
NB_R = 16
R_BLK = 40
ROWS_PAD = NB_R * R_BLK
SP = 584
N_WORKERS = 32
COL_OFFS = tuple(range(0, SP - 16, 16)) + (SP - 16,)


def _sc_gather_body(nrd, head_base, table_hbm, idx_hbm, pos_hbm,
                    table_v, idx_v, out_v, sem):
    wid = lax.axis_index("s") * 2 + lax.axis_index("c")
    pltpu.sync_copy(table_hbm, table_v)
    h_local = wid // NB_R
    rb = wid % NB_R
    r0 = rb * R_BLK
    pltpu.sync_copy(idx_hbm.at[pl.ds(r0, R_BLK), :], idx_v)
    hoff = (head_base + h_local) * nrd

    @plsc.parallel_loop(0, R_BLK, unroll=2)
    def row_body(r):
        for off in COL_OFFS:
            idx16 = idx_v[r, pl.ds(off, 16)]
            out_v[r, pl.ds(off, 16)] = plsc.load_gather(table_v, [idx16 + hoff])

    pltpu.sync_copy(out_v, pos_hbm.at[h_local, pl.ds(r0, R_BLK), :])


def _sc_gather_chunk(table_t_flat, idx_pad, nrd, head_base):
    mesh = plsc.VectorSubcoreMesh(core_axis_name="c", subcore_axis_name="s")
    fn = functools.partial(
        pl.kernel,
        mesh=mesh,
        out_type=jax.ShapeDtypeStruct((HEADS_PER_CHUNK, ROWS_PAD, SP), jnp.float32),
        scratch_types=[
            pltpu.VMEM((NUM_HEADS * nrd,), jnp.float32),
            pltpu.VMEM((R_BLK, SP), jnp.int32),
            pltpu.VMEM((R_BLK, SP), jnp.float32),
            pltpu.SemaphoreType.DMA,
        ],
        compiler_params=pltpu.CompilerParams(needs_layout_passes=False),
    )(functools.partial(_sc_gather_body, nrd, head_base))
    return fn(table_t_flat, idx_pad)


def _add_body(prev_ref, a_ref, p_ref, o_ref):
    del prev_ref
    o_ref[...] = a_ref[...] + p_ref[:, :SEQ, :SEQ][None]


def _tc_add_chunk(prev_out, attn, pos_chunk, head_base):
    b, nh, s, _ = attn.shape
    hb = head_base
    return pl.pallas_call(
        _add_body,
        grid=(HEADS_PER_CHUNK,),
        in_specs=[
            pl.BlockSpec(memory_space=pl.ANY),
            pl.BlockSpec((b, 1, s, s), lambda h: (0, hb + h, 0, 0)),
            pl.BlockSpec((1, SP, SP), lambda h: (h, 0, 0)),
        ],
        out_specs=pl.BlockSpec((b, 1, s, s), lambda h: (0, hb + h, 0, 0)),
        out_shape=jax.ShapeDtypeStruct(attn.shape, attn.dtype),
        input_output_aliases={0: 0},
        compiler_params=pltpu.CompilerParams(
            vmem_limit_bytes=110 * 1024 * 1024,
        ),
    )(prev_out, attn, pos_chunk)


def kernel(attention_scores, relative_position_bias_table, relative_position_index):
    nrd = relative_position_bias_table.shape[0]
    table_t_flat = jnp.transpose(relative_position_bias_table).reshape(-1)
    idx_pad = jnp.pad(
        relative_position_index,
        ((0, ROWS_PAD - SEQ), (0, SP - SEQ)),
    )
    pos_chunks = [
        _sc_gather_chunk(table_t_flat, idx_pad, nrd, k * HEADS_PER_CHUNK)
        for k in range(N_CHUNKS)
    ]
    out = jnp.empty(attention_scores.shape, attention_scores.dtype)
    for k in range(N_CHUNKS):
        out = _tc_add_chunk(out, attention_scores, pos_chunks[k],
                            k * HEADS_PER_CHUNK)
    return out


# speedup vs baseline: 1.2529x; 1.2529x over previous
import functools

import jax
import jax.numpy as jnp
from jax import lax
from jax.experimental import pallas as pl
from jax.experimental.pallas import tpu as pltpu
from jax.experimental.pallas import tpu_sc as plsc

SEQ = 577
NUM_HEADS = 12
NB_R = 16
R_BLK = 40
ROWS_PAD = NB_R * R_BLK
SP = 584
HEAD_HALVES = 2
HEADS_PER_HALF = NUM_HEADS // HEAD_HALVES
COL_OFFS = tuple(range(0, SP - 16, 16)) + (SP - 16,)


def _sc_gather_body(nrd, table_hbm, idx_hbm, pos_hbm,
                    table_v, idx_v, out_v0, out_v1, tsem, isem, osem0, osem1):
    out_bufs = (out_v0, out_v1)
    osems = (osem0, osem1)
    wid = lax.axis_index("s") * 2 + lax.axis_index("c")
    hh = wid // NB_R
    rb = wid % NB_R
    r0 = rb * R_BLK
    h0 = hh * HEADS_PER_HALF

    tcopy = pltpu.make_async_copy(table_hbm, table_v, tsem)
    tcopy.start()
    icopy = pltpu.make_async_copy(idx_hbm.at[pl.ds(r0, R_BLK), :], idx_v, isem)
    icopy.start()
    tcopy.wait()
    icopy.wait()

    ocopies = [None, None]
    for dh in range(HEADS_PER_HALF):
        s = dh % 2
        if ocopies[s] is not None:
            ocopies[s].wait()
        out_v = out_bufs[s]
        hoff = (h0 + dh) * nrd

        @plsc.parallel_loop(0, R_BLK, unroll=4)
        def row_body(r, out_v=out_v, hoff=hoff):
            for off in COL_OFFS:
                idx16 = idx_v[r, pl.ds(off, 16)]
                out_v[r, pl.ds(off, 16)] = plsc.load_gather(
                    table_v, [idx16 + hoff])

        ocopies[s] = pltpu.make_async_copy(
            out_v, pos_hbm.at[h0 + dh, pl.ds(r0, R_BLK), :], osems[s])
        ocopies[s].start()

    for s in range(2):
        if ocopies[s] is not None:
            ocopies[s].wait()


def _sc_gather(table_t_flat, idx_pad, nrd):
    mesh = plsc.VectorSubcoreMesh(core_axis_name="c", subcore_axis_name="s")
    fn = functools.partial(
        pl.kernel,
        mesh=mesh,
        out_type=jax.ShapeDtypeStruct((NUM_HEADS, ROWS_PAD, SP), jnp.float32),
        scratch_types=[
            pltpu.VMEM((NUM_HEADS * nrd,), jnp.float32),
            pltpu.VMEM((R_BLK, SP), jnp.int32),
            pltpu.VMEM((R_BLK, SP), jnp.float32),
            pltpu.VMEM((R_BLK, SP), jnp.float32),
            pltpu.SemaphoreType.DMA,
            pltpu.SemaphoreType.DMA,
            pltpu.SemaphoreType.DMA,
            pltpu.SemaphoreType.DMA,
        ],
        compiler_params=pltpu.CompilerParams(needs_layout_passes=False),
    )(functools.partial(_sc_gather_body, nrd))
    return fn(table_t_flat, idx_pad)


def _add_body(a_ref, p_ref, o_ref):
    o_ref[...] = a_ref[...] + p_ref[:, :SEQ, :SEQ][None]


def _tc_add(attn, pos_pad):
    b, nh, s, _ = attn.shape
    return pl.pallas_call(
        _add_body,
        grid=(nh,),
        in_specs=[
            pl.BlockSpec((b, 1, s, s), lambda h: (0, h, 0, 0)),
            pl.BlockSpec((1, SP, SP), lambda h: (h, 0, 0)),
        ],
        out_specs=pl.BlockSpec((b, 1, s, s), lambda h: (0, h, 0, 0)),
        out_shape=jax.ShapeDtypeStruct(attn.shape, attn.dtype),
        compiler_params=pltpu.CompilerParams(
            vmem_limit_bytes=110 * 1024 * 1024,
        ),
    )(attn, pos_pad)


def kernel(attention_scores, relative_position_bias_table, relative_position_index):
    nrd = relative_position_bias_table.shape[0]
    table_t_flat = jnp.transpose(relative_position_bias_table).reshape(-1)
    idx_pad = jnp.pad(
        relative_position_index,
        ((0, ROWS_PAD - SEQ), (0, SP - SEQ)),
    )
    pos_pad = _sc_gather(table_t_flat, idx_pad, nrd)
    return _tc_add(attention_scores, pos_pad)
